# baseline (speedup 1.0000x reference)
import jax
import jax.numpy as jnp
from jax.experimental import pallas as pl

B, N, C = 8, 20000, 80
GROUP = 8
LROW = B * N // GROUP
LCOL = GROUP * C
G = 5
BR = LROW // G
RCOL = N * 4
BC = RCOL // G


def _fused_kernel(logits_ref, cent_ref, regress_ref, pts_ref,
                  logits_out_ref, boxes_out_ref):
    l = logits_ref[...]
    c = cent_ref[...]
    lane = jax.lax.broadcasted_iota(jnp.int32, l.shape, 1)
    idx = lane // C
    cb = c[:, 0:1]
    for j in range(1, GROUP):
        cb = jnp.where(idx == j, c[:, j:j + 1], cb)
    logits_out_ref[...] = jnp.sqrt(l * cb)

    r = regress_ref[...]
    rlane = jax.lax.broadcasted_iota(jnp.int32, r.shape, 1)
    sign = jnp.where((rlane % 4) >= 2, 1.0, -1.0).astype(jnp.float32)
    boxes = pts_ref[...] + r * sign
    boxes_out_ref[...] = jnp.clip(boxes, 0.0, 1.0)


def kernel(logits, regress, points, centerness):
    logits2 = logits.reshape(LROW, LCOL)
    cent2 = centerness.reshape(LROW, GROUP)
    regress2 = regress.reshape(B, RCOL)
    pts_flat = jnp.stack(
        [points[:, 0], points[:, 1], points[:, 0], points[:, 1]], axis=1
    ).reshape(1, RCOL)

    out = pl.pallas_call(
        _fused_kernel,
        grid=(G,),
        in_specs=[
            pl.BlockSpec((BR, LCOL), lambda i: (i, 0)),
            pl.BlockSpec((BR, GROUP), lambda i: (i, 0)),
            pl.BlockSpec((B, BC), lambda i: (0, i)),
            pl.BlockSpec((1, BC), lambda i: (0, i)),
        ],
        out_specs=[
            pl.BlockSpec((BR, LCOL), lambda i: (i, 0)),
            pl.BlockSpec((B, BC), lambda i: (0, i)),
        ],
        out_shape=[
            jax.ShapeDtypeStruct((LROW, LCOL), jnp.float32),
            jax.ShapeDtypeStruct((B, RCOL), jnp.float32),
        ],
    )(logits2, cent2, regress2, pts_flat)
    return (out[0].reshape(B, N, C), out[1].reshape(B, N, 4))


# speedup vs baseline: 13.4734x; 13.4734x over previous
import jax
import jax.numpy as jnp
from jax.experimental import pallas as pl

B, N, C = 8, 20000, 80


def _fused_kernel(logits_ref, cent_ref, regress_ref, pts_ref,
                  logits_out_ref, boxes_out_ref):
    l = logits_ref[...]
    c = cent_ref[...]
    logits_out_ref[...] = jnp.sqrt(l * c)

    r = regress_ref[...]
    px = pts_ref[0:1, :][None]
    py = pts_ref[1:2, :][None]
    row = jax.lax.broadcasted_iota(jnp.int32, r.shape, 1)
    sign = jnp.where(row >= 2, 1.0, -1.0).astype(jnp.float32)
    pts4 = jnp.where(row % 2 == 0, px, py)
    boxes_out_ref[...] = jnp.clip(pts4 + sign * r, 0.0, 1.0)


def kernel(logits, regress, points, centerness):
    lt = jnp.transpose(logits, (0, 2, 1))
    rt = jnp.transpose(regress, (0, 2, 1))
    pt = jnp.transpose(points, (1, 0))
    ct = jnp.transpose(centerness, (0, 2, 1))

    out = pl.pallas_call(
        _fused_kernel,
        grid=(B,),
        in_specs=[
            pl.BlockSpec((1, C, N), lambda b: (b, 0, 0)),
            pl.BlockSpec((1, 1, N), lambda b: (b, 0, 0)),
            pl.BlockSpec((1, 4, N), lambda b: (b, 0, 0)),
            pl.BlockSpec((2, N), lambda b: (0, 0)),
        ],
        out_specs=[
            pl.BlockSpec((1, C, N), lambda b: (b, 0, 0)),
            pl.BlockSpec((1, 4, N), lambda b: (b, 0, 0)),
        ],
        out_shape=[
            jax.ShapeDtypeStruct((B, C, N), jnp.float32),
            jax.ShapeDtypeStruct((B, 4, N), jnp.float32),
        ],
    )(lt, ct, rt, pt)
    return (jnp.transpose(out[0], (0, 2, 1)), jnp.transpose(out[1], (0, 2, 1)))
